# initial kernel scaffold (unmeasured)
import functools

import jax
import jax.numpy as jnp
from jax import lax
from jax.experimental import pallas as pl
from jax.experimental.pallas import tpu as pltpu

N = 16
M = 64
D = 1024
R = M // N


def kernel(x, Win0, Wout0, Win1, Wout1, Win2, Wout2):
    def body(x_ref, win0_ref, wout0_ref, win1_ref, wout1_ref,
             win2_ref, wout2_ref, out_ref,
             part_ref, y_ref, rs_ref, ag_ref,
             send_sems, recv_rs, recv_ag):
        my = lax.axis_index("i")

        barrier = pltpu.get_barrier_semaphore()
        for p in range(N):
            @pl.when(p != my)
            def _():
                pl.semaphore_signal(
                    barrier, inc=1, device_id=(p,),
                    device_id_type=pl.DeviceIdType.MESH,
                )
        pl.semaphore_wait(barrier, N - 1)

        def rdma(src, dst, ssem, rsem, target):
            return pltpu.make_async_remote_copy(
                src_ref=src, dst_ref=dst, send_sem=ssem, recv_sem=rsem,
                device_id=(target,), device_id_type=pl.DeviceIdType.MESH,
            )

        def all_to_all(src_slot_fn, dst_buf, recv_sem_arr, own_value):
            for p in range(N):
                @pl.when(p == my)
                def _():
                    dst_buf[p] = own_value
                @pl.when(p != my)
                def _():
                    rdma(src_slot_fn(p), dst_buf.at[my],
                         send_sems.at[p], recv_sem_arr.at[my], p).start()
            for p in range(N):
                @pl.when(p != my)
                def _():
                    rdma(src_slot_fn(p), dst_buf.at[my],
                         send_sems.at[p], recv_sem_arr.at[my], p).wait_send()
            for p in range(N):
                @pl.when(p != my)
                def _():
                    rdma(src_slot_fn(p), dst_buf.at[p],
                         send_sems.at[p], recv_sem_arr.at[p], p).wait_recv()

        xb = x_ref[...].astype(jnp.bfloat16)
        weight_refs = [(win0_ref, wout0_ref), (win1_ref, wout1_ref),
                       (win2_ref, wout2_ref)]

        for l, (win_ref, wout_ref) in enumerate(weight_refs):
            wi = win_ref[...].astype(jnp.bfloat16)
            h = jnp.dot(xb, wi, preferred_element_type=jnp.float32)
            hb = jnp.maximum(h, 0.0).astype(jnp.bfloat16)
            wo = wout_ref[...].astype(jnp.bfloat16)
            partial = jnp.dot(hb, wo, preferred_element_type=jnp.float32)
            part_ref[...] = partial.reshape(N, R, D).astype(jnp.bfloat16)

            all_to_all(lambda p: part_ref.at[p], rs_ref, recv_rs,
                       part_ref[my])
            y = jnp.sum(rs_ref[...].astype(jnp.float32), axis=0)

            if l < 2:
                y_ref[...] = y.astype(jnp.bfloat16)
                all_to_all(lambda p: y_ref, ag_ref, recv_ag, y_ref[...])
                xb = ag_ref[...].reshape(M, D)
            else:
                out_ref[...] = y

    return pl.pallas_call(
        body,
        out_shape=jax.ShapeDtypeStruct((R, D), jnp.float32),
        in_specs=[pl.BlockSpec(memory_space=pltpu.VMEM)] * 7,
        out_specs=pl.BlockSpec(memory_space=pltpu.VMEM),
        scratch_shapes=[
            pltpu.VMEM((N, R, D), jnp.bfloat16),
            pltpu.VMEM((R, D), jnp.bfloat16),
            pltpu.VMEM((N, R, D), jnp.bfloat16),
            pltpu.VMEM((N, R, D), jnp.bfloat16),
            pltpu.SemaphoreType.DMA((N,)),
            pltpu.SemaphoreType.DMA((N,)),
            pltpu.SemaphoreType.DMA((N,)),
        ],
        compiler_params=pltpu.CompilerParams(collective_id=0),
    )(x, Win0, Wout0, Win1, Wout1, Win2, Wout2)


# baseline (device time: 51615 ns/iter reference)
import functools

import jax
import jax.numpy as jnp
from jax import lax
from jax.experimental import pallas as pl
from jax.experimental.pallas import tpu as pltpu

N = 16
M = 64
D = 1024
R = M // N


def kernel(x, Win0, Wout0, Win1, Wout1, Win2, Wout2):
    def body(x_ref, win0_ref, wout0_ref, win1_ref, wout1_ref,
             win2_ref, wout2_ref, out_ref,
             win_buf, wout_buf, part_ref, y_ref, rs_ref, ag_ref,
             send_sems, recv_rs, recv_ag, load_sems):
        my = lax.axis_index("i")

        barrier = pltpu.get_barrier_semaphore()
        for p in range(N):
            @pl.when(p != my)
            def _():
                pl.semaphore_signal(
                    barrier, inc=1, device_id=(p,),
                    device_id_type=pl.DeviceIdType.MESH,
                )
        pl.semaphore_wait(barrier, N - 1)

        def rdma(src, dst, ssem, rsem, target):
            return pltpu.make_async_remote_copy(
                src_ref=src, dst_ref=dst, send_sem=ssem, recv_sem=rsem,
                device_id=(target,), device_id_type=pl.DeviceIdType.MESH,
            )

        def all_to_all(src_slot_fn, dst_buf, recv_sem_arr, own_value):
            for p in range(N):
                @pl.when(p == my)
                def _():
                    dst_buf[p] = own_value
                @pl.when(p != my)
                def _():
                    rdma(src_slot_fn(p), dst_buf.at[my],
                         send_sems.at[p], recv_sem_arr.at[my], p).start()
            for p in range(N):
                @pl.when(p != my)
                def _():
                    rdma(src_slot_fn(p), dst_buf.at[my],
                         send_sems.at[p], recv_sem_arr.at[my], p).wait_send()
            for p in range(N):
                @pl.when(p != my)
                def _():
                    rdma(src_slot_fn(p), dst_buf.at[p],
                         send_sems.at[p], recv_sem_arr.at[p], p).wait_recv()

        xb = x_ref[...].astype(jnp.bfloat16)
        weight_refs = [(win0_ref, wout0_ref), (win1_ref, wout1_ref),
                       (win2_ref, wout2_ref)]

        for l, (win_ref, wout_ref) in enumerate(weight_refs):
            cp_in = pltpu.make_async_copy(win_ref, win_buf, load_sems.at[0])
            cp_out = pltpu.make_async_copy(wout_ref, wout_buf, load_sems.at[1])
            cp_in.start()
            cp_out.start()
            cp_in.wait()
            wi = win_buf[...].astype(jnp.bfloat16)
            h = jnp.dot(xb, wi, preferred_element_type=jnp.float32)
            hb = jnp.maximum(h, 0.0).astype(jnp.bfloat16)
            cp_out.wait()
            wo = wout_buf[...].astype(jnp.bfloat16)
            partial = jnp.dot(hb, wo, preferred_element_type=jnp.float32)
            part_ref[...] = partial.reshape(N, R, D).astype(jnp.bfloat16)

            all_to_all(lambda p: part_ref.at[p], rs_ref, recv_rs,
                       part_ref[my])
            y = jnp.sum(rs_ref[...].astype(jnp.float32), axis=0)

            if l < 2:
                y_ref[...] = y.astype(jnp.bfloat16)
                all_to_all(lambda p: y_ref, ag_ref, recv_ag, y_ref[...])
                xb = ag_ref[...].reshape(M, D)
            else:
                out_ref[...] = y

    return pl.pallas_call(
        body,
        out_shape=jax.ShapeDtypeStruct((R, D), jnp.float32),
        in_specs=[pl.BlockSpec(memory_space=pltpu.VMEM)]
        + [pl.BlockSpec(memory_space=pl.ANY)] * 6,
        out_specs=pl.BlockSpec(memory_space=pltpu.VMEM),
        scratch_shapes=[
            pltpu.VMEM((D, 2 * D), jnp.float32),
            pltpu.VMEM((2 * D, D), jnp.float32),
            pltpu.VMEM((N, R, D), jnp.bfloat16),
            pltpu.VMEM((R, D), jnp.bfloat16),
            pltpu.VMEM((N, R, D), jnp.bfloat16),
            pltpu.VMEM((N, R, D), jnp.bfloat16),
            pltpu.SemaphoreType.DMA((N,)),
            pltpu.SemaphoreType.DMA((N,)),
            pltpu.SemaphoreType.DMA((N,)),
            pltpu.SemaphoreType.DMA((2,)),
        ],
        compiler_params=pltpu.CompilerParams(collective_id=0),
    )(x, Win0, Wout0, Win1, Wout1, Win2, Wout2)


# device time: 41963 ns/iter; 1.2300x vs baseline; 1.2300x over previous
import functools

import jax
import jax.numpy as jnp
from jax import lax
from jax.experimental import pallas as pl
from jax.experimental.pallas import tpu as pltpu

N = 16
M = 64
D = 1024
R = M // N


def kernel(x, Win0, Wout0, Win1, Wout1, Win2, Wout2):
    def body(x_ref, win0_ref, wout0_ref, win1_ref, wout1_ref,
             win2_ref, wout2_ref, out_ref,
             win_buf, wout_buf, part_ref, y_ref, rs_ref, ag_ref,
             send_sems, recv_rs, recv_ag, load_sems):
        my = lax.axis_index("i")

        weight_hbm = [(win0_ref, wout0_ref), (win1_ref, wout1_ref),
                      (win2_ref, wout2_ref)]

        def load_win(l):
            return pltpu.make_async_copy(weight_hbm[l][0], win_buf,
                                         load_sems.at[0])

        def load_wout(l):
            return pltpu.make_async_copy(weight_hbm[l][1], wout_buf,
                                         load_sems.at[1])

        load_win(0).start()
        load_wout(0).start()

        barrier = pltpu.get_barrier_semaphore()
        for p in range(N):
            @pl.when(p != my)
            def _():
                pl.semaphore_signal(
                    barrier, inc=1, device_id=(p,),
                    device_id_type=pl.DeviceIdType.MESH,
                )
        pl.semaphore_wait(barrier, N - 1)

        def rdma(src, dst, ssem, rsem, target):
            return pltpu.make_async_remote_copy(
                src_ref=src, dst_ref=dst, send_sem=ssem, recv_sem=rsem,
                device_id=(target,), device_id_type=pl.DeviceIdType.MESH,
            )

        def all_to_all(src_slot_fn, dst_buf, recv_sem_arr, own_value):
            for p in range(N):
                @pl.when(p == my)
                def _():
                    dst_buf[p] = own_value
                @pl.when(p != my)
                def _():
                    rdma(src_slot_fn(p), dst_buf.at[my],
                         send_sems.at[p], recv_sem_arr.at[my], p).start()
            for p in range(N):
                @pl.when(p != my)
                def _():
                    rdma(src_slot_fn(p), dst_buf.at[my],
                         send_sems.at[p], recv_sem_arr.at[my], p).wait_send()
            for p in range(N):
                @pl.when(p != my)
                def _():
                    rdma(src_slot_fn(p), dst_buf.at[p],
                         send_sems.at[p], recv_sem_arr.at[p], p).wait_recv()

        xb = x_ref[...].astype(jnp.bfloat16)

        for l in range(3):
            load_win(l).wait()
            wi = win_buf[...].astype(jnp.bfloat16)
            h = jnp.dot(xb, wi, preferred_element_type=jnp.float32)
            hb = jnp.maximum(h, 0.0).astype(jnp.bfloat16)
            if l < 2:
                load_win(l + 1).start()
            load_wout(l).wait()
            wo = wout_buf[...].astype(jnp.bfloat16)
            partial = jnp.dot(hb, wo, preferred_element_type=jnp.float32)
            if l < 2:
                load_wout(l + 1).start()
            part_ref[...] = partial.reshape(N, R, D).astype(jnp.bfloat16)

            all_to_all(lambda p: part_ref.at[p], rs_ref, recv_rs,
                       part_ref[my])
            y = jnp.sum(rs_ref[...].astype(jnp.float32), axis=0)

            if l < 2:
                y_ref[...] = y.astype(jnp.bfloat16)
                all_to_all(lambda p: y_ref, ag_ref, recv_ag, y_ref[...])
                xb = ag_ref[...].reshape(M, D)
            else:
                out_ref[...] = y

    return pl.pallas_call(
        body,
        out_shape=jax.ShapeDtypeStruct((R, D), jnp.float32),
        in_specs=[pl.BlockSpec(memory_space=pltpu.VMEM)]
        + [pl.BlockSpec(memory_space=pl.ANY)] * 6,
        out_specs=pl.BlockSpec(memory_space=pltpu.VMEM),
        scratch_shapes=[
            pltpu.VMEM((D, 2 * D), jnp.float32),
            pltpu.VMEM((2 * D, D), jnp.float32),
            pltpu.VMEM((N, R, D), jnp.bfloat16),
            pltpu.VMEM((R, D), jnp.bfloat16),
            pltpu.VMEM((N, R, D), jnp.bfloat16),
            pltpu.VMEM((N, R, D), jnp.bfloat16),
            pltpu.SemaphoreType.DMA((N,)),
            pltpu.SemaphoreType.DMA((N,)),
            pltpu.SemaphoreType.DMA((N,)),
            pltpu.SemaphoreType.DMA((2,)),
        ],
        compiler_params=pltpu.CompilerParams(collective_id=0),
    )(x, Win0, Wout0, Win1, Wout1, Win2, Wout2)


# device time: 41927 ns/iter; 1.2311x vs baseline; 1.0009x over previous
import jax
import jax.numpy as jnp
from jax import lax
from jax.experimental import pallas as pl
from jax.experimental.pallas import tpu as pltpu

N = 16
M = 64
D = 1024
R = M // N
C = 2
DC = D // C


def kernel(x, Win0, Wout0, Win1, Wout1, Win2, Wout2):
    f32 = jnp.float32
    bf16 = jnp.bfloat16

    def body(x_ref, win0_ref, wout0_ref, win1_ref, wout1_ref,
             win2_ref, wout2_ref, out_ref,
             win_buf, wout_buf, part_ref, y_ref, rs_ref, ag_ref,
             send_sems, recv_rs, recv_ag, load_sems):
        my = lax.axis_index("i")

        weight_hbm = [(win0_ref, wout0_ref), (win1_ref, wout1_ref),
                      (win2_ref, wout2_ref)]

        def load_win(l):
            return pltpu.make_async_copy(weight_hbm[l][0], win_buf,
                                         load_sems.at[0])

        def load_wout(l):
            return pltpu.make_async_copy(weight_hbm[l][1], wout_buf,
                                         load_sems.at[1])

        load_win(0).start()
        load_wout(0).start()

        barrier = pltpu.get_barrier_semaphore()
        for p in range(N):
            @pl.when(p != my)
            def _():
                pl.semaphore_signal(
                    barrier, inc=1, device_id=(p,),
                    device_id_type=pl.DeviceIdType.MESH,
                )
        pl.semaphore_wait(barrier, N - 1)

        def rdma(src, dst, ssem, rsem, target):
            return pltpu.make_async_remote_copy(
                src_ref=src, dst_ref=dst, send_sem=ssem, recv_sem=rsem,
                device_id=(target,), device_id_type=pl.DeviceIdType.MESH,
            )

        sent_before = [False, False]

        def drain_sends(c):
            if sent_before[c]:
                for p in range(N):
                    @pl.when(p != my)
                    def _():
                        rdma(part_ref.at[c, 0], rs_ref.at[c, 0],
                             send_sems.at[c, p], recv_rs.at[c, p],
                             p).wait_send()
            sent_before[c] = True

        def send_chunk(c, src_slot_fn, dst_buf, rsem):
            drain_sends(c)
            for p in range(N):
                @pl.when(p == my)
                def _():
                    dst_buf[c, p] = src_slot_fn(p)[...]
                @pl.when(p != my)
                def _():
                    rdma(src_slot_fn(p), dst_buf.at[c, my],
                         send_sems.at[c, p], rsem.at[c, my], p).start()

        def wait_chunk(c, dst_buf, rsem):
            for p in range(N):
                @pl.when(p != my)
                def _():
                    rdma(part_ref.at[c, 0], dst_buf.at[c, p],
                         send_sems.at[c, p], rsem.at[c, p], p).wait_recv()

        xc = [x_ref[:, c * DC:(c + 1) * DC].astype(bf16) for c in range(C)]
        load_win(0).wait()
        wi = win_buf[...].astype(bf16)
        acc = sum(jnp.dot(xc[c], wi[c * DC:(c + 1) * DC],
                          preferred_element_type=f32) for c in range(C))
        hb = jnp.maximum(acc, 0.0).astype(bf16)
        load_win(1).start()

        for l in range(3):
            load_wout(l).wait()
            wo = wout_buf[...].astype(bf16)
            for c in range(C):
                pc = jnp.dot(hb, wo[:, c * DC:(c + 1) * DC],
                             preferred_element_type=f32)
                part_ref[c] = pc.reshape(N, R, DC).astype(bf16)
                send_chunk(c, lambda p: part_ref.at[c, p], rs_ref, recv_rs)
            if l < 2:
                load_wout(l + 1).start()

            for c in range(C):
                wait_chunk(c, rs_ref, recv_rs)
                y = jnp.sum(rs_ref[c].astype(f32), axis=0)
                if l < 2:
                    y_ref[c] = y.astype(bf16)
                    send_chunk(c, lambda p: y_ref.at[c], ag_ref, recv_ag)
                else:
                    out_ref[:, c * DC:(c + 1) * DC] = y

            if l < 2:
                load_win(l + 1).wait()
                wi = win_buf[...].astype(bf16)
                if l == 0:
                    load_win(2).start()
                acc = None
                for c in range(C):
                    wait_chunk(c, ag_ref, recv_ag)
                    xcb = ag_ref[c].reshape(M, DC)
                    t = jnp.dot(xcb, wi[c * DC:(c + 1) * DC],
                                preferred_element_type=f32)
                    acc = t if acc is None else acc + t
                hb = jnp.maximum(acc, 0.0).astype(bf16)

        for c in range(C):
            drain_sends(c)

    return pl.pallas_call(
        body,
        out_shape=jax.ShapeDtypeStruct((R, D), f32),
        in_specs=[pl.BlockSpec(memory_space=pltpu.VMEM)]
        + [pl.BlockSpec(memory_space=pl.ANY)] * 6,
        out_specs=pl.BlockSpec(memory_space=pltpu.VMEM),
        scratch_shapes=[
            pltpu.VMEM((D, 2 * D), f32),
            pltpu.VMEM((2 * D, D), f32),
            pltpu.VMEM((C, N, R, DC), bf16),
            pltpu.VMEM((C, R, DC), bf16),
            pltpu.VMEM((C, N, R, DC), bf16),
            pltpu.VMEM((C, N, R, DC), bf16),
            pltpu.SemaphoreType.DMA((C, N)),
            pltpu.SemaphoreType.DMA((C, N)),
            pltpu.SemaphoreType.DMA((C, N)),
            pltpu.SemaphoreType.DMA((2,)),
        ],
        compiler_params=pltpu.CompilerParams(collective_id=0),
    )(x, Win0, Wout0, Win1, Wout1, Win2, Wout2)


# device time: 29652 ns/iter; 1.7407x vs baseline; 1.4140x over previous
import jax
import jax.numpy as jnp
from jax import lax
from jax.experimental import pallas as pl
from jax.experimental.pallas import tpu as pltpu

import os
ABLATE = os.environ.get("KERNEL_ABLATE", "")

N = 16
M = 64
D = 1024
R = M // N
C = 2
DC = D // C


def kernel(x, Win0, Wout0, Win1, Wout1, Win2, Wout2):
    f32 = jnp.float32
    bf16 = jnp.bfloat16

    def body(x_ref, win0_ref, wout0_ref, win1_ref, wout1_ref,
             win2_ref, wout2_ref, out_ref,
             win_buf, wout_buf, part_ref, y_ref, rs_ref, ag_ref,
             send_sems, recv_rs, recv_ag, load_sems):
        my = lax.axis_index("i")

        weight_hbm = [(win0_ref, wout0_ref), (win1_ref, wout1_ref),
                      (win2_ref, wout2_ref)]

        def load_win(l):
            return pltpu.make_async_copy(weight_hbm[l][0], win_buf,
                                         load_sems.at[0])

        def load_wout(l):
            return pltpu.make_async_copy(weight_hbm[l][1], wout_buf,
                                         load_sems.at[1])

        load_win(0).start()
        load_wout(0).start()

        barrier = pltpu.get_barrier_semaphore()
        for p in range(N):
            @pl.when(p != my)
            def _():
                pl.semaphore_signal(
                    barrier, inc=1, device_id=(p,),
                    device_id_type=pl.DeviceIdType.MESH,
                )
        pl.semaphore_wait(barrier, N - 1)

        def rdma(src, dst, ssem, rsem, target):
            return pltpu.make_async_remote_copy(
                src_ref=src, dst_ref=dst, send_sem=ssem, recv_sem=rsem,
                device_id=(target,), device_id_type=pl.DeviceIdType.MESH,
            )

        sent_before = [False, False]

        def drain_sends(c):
            if sent_before[c]:
                for p in range(N):
                    @pl.when(p != my)
                    def _():
                        rdma(part_ref.at[c, 0], rs_ref.at[c, 0],
                             send_sems.at[c, p], recv_rs.at[c, p],
                             p).wait_send()
            sent_before[c] = True

        def send_chunk(c, src_slot_fn, dst_buf, rsem):
            drain_sends(c)
            for p in range(N):
                @pl.when(p == my)
                def _():
                    dst_buf[c, p] = src_slot_fn(p)[...]
                @pl.when(p != my)
                def _():
                    rdma(src_slot_fn(p), dst_buf.at[c, my],
                         send_sems.at[c, p], rsem.at[c, my], p).start()

        def wait_chunk(c, dst_buf, rsem):
            for p in range(N):
                @pl.when(p != my)
                def _():
                    rdma(part_ref.at[c, 0], dst_buf.at[c, p],
                         send_sems.at[c, p], rsem.at[c, p], p).wait_recv()

        xc = [x_ref[:, c * DC:(c + 1) * DC].astype(bf16) for c in range(C)]
        load_win(0).wait()
        wi = win_buf[...].astype(bf16)
        acc = sum(jnp.dot(xc[c], wi[c * DC:(c + 1) * DC],
                          preferred_element_type=f32) for c in range(C))
        hb = jnp.maximum(acc, 0.0).astype(bf16)
        load_win(1).start()

        if ABLATE == "mm_only":
            for l in range(3):
                load_wout(l).wait()
                wo = wout_buf[...].astype(bf16)
                pc = [jnp.dot(hb, wo[:, c * DC:(c + 1) * DC],
                              preferred_element_type=f32) for c in range(C)]
                if l < 2:
                    load_wout(l + 1).start()
                    load_win(l + 1).wait()
                    wi = win_buf[...].astype(bf16)
                    if l == 0:
                        load_win(2).start()
                    acc = sum(jnp.dot(pc[c].astype(bf16),
                                      wi[c * DC:(c + 1) * DC],
                                      preferred_element_type=f32)
                              for c in range(C))
                    hb = jnp.maximum(acc, 0.0).astype(bf16)
                else:
                    out_ref[...] = jnp.concatenate(
                        [p[:R] for p in pc], axis=1)
            return

        for l in range(3):
            load_wout(l).wait()
            wo = wout_buf[...].astype(bf16)
            for c in range(C):
                pc = jnp.dot(hb, wo[:, c * DC:(c + 1) * DC],
                             preferred_element_type=f32)
                part_ref[c] = pc.reshape(N, R, DC).astype(bf16)
                if ABLATE != "compute_only":
                    send_chunk(c, lambda p: part_ref.at[c, p], rs_ref, recv_rs)
            if l < 2:
                load_wout(l + 1).start()

            for c in range(C):
                if ABLATE != "compute_only":
                    wait_chunk(c, rs_ref, recv_rs)
                    y = jnp.sum(rs_ref[c].astype(f32), axis=0)
                else:
                    y = jnp.sum(part_ref[c].astype(f32), axis=0)
                if l < 2:
                    y_ref[c] = y.astype(bf16)
                    if ABLATE != "compute_only":
                        send_chunk(c, lambda p: y_ref.at[c], ag_ref, recv_ag)
                else:
                    out_ref[:, c * DC:(c + 1) * DC] = y

            if l < 2:
                load_win(l + 1).wait()
                wi = win_buf[...].astype(bf16)
                if l == 0:
                    load_win(2).start()
                acc = None
                for c in range(C):
                    if ABLATE != "compute_only":
                        wait_chunk(c, ag_ref, recv_ag)
                        xcb = ag_ref[c].reshape(M, DC)
                    else:
                        xcb = part_ref[c].reshape(M, DC)
                    t = jnp.dot(xcb, wi[c * DC:(c + 1) * DC],
                                preferred_element_type=f32)
                    acc = t if acc is None else acc + t
                hb = jnp.maximum(acc, 0.0).astype(bf16)

        for c in range(C):
            drain_sends(c)

    return pl.pallas_call(
        body,
        out_shape=jax.ShapeDtypeStruct((R, D), f32),
        in_specs=[pl.BlockSpec(memory_space=pltpu.VMEM)]
        + [pl.BlockSpec(memory_space=pl.ANY)] * 6,
        out_specs=pl.BlockSpec(memory_space=pltpu.VMEM),
        scratch_shapes=[
            pltpu.VMEM((D, 2 * D), f32),
            pltpu.VMEM((2 * D, D), f32),
            pltpu.VMEM((C, N, R, DC), bf16),
            pltpu.VMEM((C, R, DC), bf16),
            pltpu.VMEM((C, N, R, DC), bf16),
            pltpu.VMEM((C, N, R, DC), bf16),
            pltpu.SemaphoreType.DMA((C, N)),
            pltpu.SemaphoreType.DMA((C, N)),
            pltpu.SemaphoreType.DMA((C, N)),
            pltpu.SemaphoreType.DMA((2,)),
        ],
        compiler_params=pltpu.CompilerParams(collective_id=0),
    )(x, Win0, Wout0, Win1, Wout1, Win2, Wout2)
